# baseline (speedup 1.0000x reference)
import functools

import jax
import jax.numpy as jnp
from jax.experimental import pallas as pl
from jax.experimental.pallas import tpu as pltpu

_N = 4096
_D_IN = 256
_D_OUT = 256
_BM = 512


def _fused_body(inputs_ref, weight_ref, adj_a_ref, adj_b_ref, bias_ref, out_ref, support_ref):
    @pl.when(pl.program_id(0) == 0)
    def _():
        support_ref[...] = jnp.dot(
            inputs_ref[...], weight_ref[...], preferred_element_type=jnp.float32
        )

    s = support_ref[...].astype(jnp.bfloat16)
    a = adj_a_ref[...].astype(jnp.bfloat16)
    b = adj_b_ref[...].astype(jnp.bfloat16)
    out_ref[:_BM] = jnp.dot(a, s, preferred_element_type=jnp.float32) + bias_ref[...]
    out_ref[_BM:] = jnp.dot(b, s, preferred_element_type=jnp.float32) + bias_ref[...]


def kernel(inputs, adj, weight, bias):
    bias2d = bias.reshape(1, _D_OUT)
    grid = (_N // (2 * _BM),)
    out = pl.pallas_call(
        _fused_body,
        grid=grid,
        in_specs=[
            pl.BlockSpec((_N, _D_IN), lambda i: (0, 0)),
            pl.BlockSpec((_D_IN, _D_OUT), lambda i: (0, 0)),
            pl.BlockSpec((_BM, _N), lambda i: (2 * i, 0)),
            pl.BlockSpec((_BM, _N), lambda i: (2 * i + 1, 0)),
            pl.BlockSpec((1, _D_OUT), lambda i: (0, 0)),
        ],
        out_specs=pl.BlockSpec((2 * _BM, _D_OUT), lambda i: (i, 0)),
        out_shape=jax.ShapeDtypeStruct((_N, _D_OUT), jnp.float32),
        scratch_shapes=[pltpu.VMEM((_N, _D_OUT), jnp.float32)],
    )(inputs, weight, adj, adj, bias2d)
    return out


# speedup vs baseline: 1.0414x; 1.0414x over previous
import functools

import jax
import jax.numpy as jnp
from jax.experimental import pallas as pl
from jax.experimental.pallas import tpu as pltpu

_N = 4096
_D_IN = 256
_D_OUT = 256
_BM = 512


def _fused_body(inputs_ref, weight_ref, adj_ref, bias_ref, out_ref, support_ref):
    @pl.when(pl.program_id(0) == 0)
    def _():
        sup = jnp.dot(
            inputs_ref[...].astype(jnp.bfloat16),
            weight_ref[...].astype(jnp.bfloat16),
            preferred_element_type=jnp.float32,
        )
        support_ref[...] = sup.astype(jnp.bfloat16)

    a = adj_ref[...].astype(jnp.bfloat16)
    acc = jnp.dot(a, support_ref[...], preferred_element_type=jnp.float32)
    out_ref[...] = acc + bias_ref[...]


def kernel(inputs, adj, weight, bias):
    bias2d = bias.reshape(1, _D_OUT)
    grid = (_N // _BM,)
    out = pl.pallas_call(
        _fused_body,
        grid=grid,
        in_specs=[
            pl.BlockSpec((_N, _D_IN), lambda i: (0, 0)),
            pl.BlockSpec((_D_IN, _D_OUT), lambda i: (0, 0)),
            pl.BlockSpec((_BM, _N), lambda i: (i, 0)),
            pl.BlockSpec((1, _D_OUT), lambda i: (0, 0)),
        ],
        out_specs=pl.BlockSpec((_BM, _D_OUT), lambda i: (i, 0)),
        out_shape=jax.ShapeDtypeStruct((_N, _D_OUT), jnp.float32),
        scratch_shapes=[pltpu.VMEM((_N, _D_OUT), jnp.bfloat16)],
    )(inputs, weight, adj, bias2d)
    return out
